# baseline (speedup 1.0000x reference)
import functools

import jax
import jax.numpy as jnp
from jax import lax
from jax.experimental import pallas as pl
from jax.experimental.pallas import tpu as pltpu
from jax.experimental.pallas import tpu_sc as plsc

D = 64
B = 16384
NC = 2
NS = 16
L = 16
NW = NC * NS
BPW = B // NW
CHUNK = 128
NCHUNK = BPW // CHUNK

_mesh = plsc.VectorSubcoreMesh(core_axis_name="c", subcore_axis_name="s")


@functools.partial(
    pl.kernel,
    mesh=_mesh,
    compiler_params=pltpu.CompilerParams(
        needs_layout_passes=False, use_tc_tiling_on_sc=False),
    out_type=jax.ShapeDtypeStruct((B,), jnp.float32),
    scratch_types=[
        pltpu.VMEM((NCHUNK, CHUNK), jnp.int32),
        pltpu.VMEM((NCHUNK, CHUNK), jnp.int32),
        pltpu.VMEM((BPW, D), jnp.float32),
        pltpu.VMEM((BPW, D), jnp.float32),
        pltpu.VMEM((BPW,), jnp.float32),
        pltpu.SemaphoreType.DMA,
        pltpu.SemaphoreType.DMA,
    ],
)
def _glove_sc(w_i_hbm, w_j_hbm, table_hbm, out_hbm,
              idx_i_v, idx_j_v, rows_i_v, rows_j_v, out_v, sem_i, sem_j):
    wid = lax.axis_index("s") * NC + lax.axis_index("c")
    base = wid * BPW

    pltpu.sync_copy(w_i_hbm.at[pl.ds(wid * NCHUNK, NCHUNK)], idx_i_v)
    pltpu.sync_copy(w_j_hbm.at[pl.ds(wid * NCHUNK, NCHUNK)], idx_j_v)

    copies = []
    for k in range(NCHUNK):
        copies.append(pltpu.async_copy(
            table_hbm.at[idx_i_v.at[k]],
            rows_i_v.at[pl.ds(k * CHUNK, CHUNK)], sem_i))
        copies.append(pltpu.async_copy(
            table_hbm.at[idx_j_v.at[k]],
            rows_j_v.at[pl.ds(k * CHUNK, CHUNK)], sem_j))
    for cp in copies:
        cp.wait()

    lane = lax.iota(jnp.int32, L)

    def group_body(g, carry):
        row0 = g * L
        acc = jnp.zeros((L,), jnp.float32)
        for b in range(L):
            rb = row0 + b
            s = (rows_i_v[rb, pl.ds(0, L)] * rows_j_v[rb, pl.ds(0, L)]
                 + rows_i_v[rb, pl.ds(L, L)] * rows_j_v[rb, pl.ds(L, L)]
                 + rows_i_v[rb, pl.ds(2 * L, L)] * rows_j_v[rb, pl.ds(2 * L, L)]
                 + rows_i_v[rb, pl.ds(3 * L, L)] * rows_j_v[rb, pl.ds(3 * L, L)])
            acc = jnp.where(lane == b, jnp.sum(s), acc)
        out_v[pl.ds(row0, L)] = acc
        return carry

    lax.fori_loop(0, BPW // L, group_body, 0)

    pltpu.sync_copy(out_v, out_hbm.at[pl.ds(base, BPW)])


def kernel(w_i, w_j, W):
    w_i = w_i.astype(jnp.int32).reshape(NW * NCHUNK, CHUNK)
    w_j = w_j.astype(jnp.int32).reshape(NW * NCHUNK, CHUNK)
    out = _glove_sc(w_i, w_j, W)
    return out.reshape(B, 1)


# speedup vs baseline: 1.8990x; 1.8990x over previous
import functools

import jax
import jax.numpy as jnp
from jax import lax
from jax.experimental import pallas as pl
from jax.experimental.pallas import tpu as pltpu
from jax.experimental.pallas import tpu_sc as plsc

D = 64
B = 16384
NC = 2
NS = 16
L = 16
NW = NC * NS
BPW = B // NW
CW = 512
MAIN = 999936
NCHUNKS = MAIN // CW
CPW = -(-NCHUNKS // NW)
CAPH = 4096
RING = 8
EROWS = 2 * B + L
DUMP = 2 * B

_mesh = plsc.VectorSubcoreMesh(core_axis_name="c", subcore_axis_name="s")
_params = pltpu.CompilerParams(needs_layout_passes=False)


@functools.partial(
    pl.kernel,
    mesh=_mesh,
    compiler_params=_params,
    out_type=jax.ShapeDtypeStruct((EROWS, 2 * D), jnp.float32),
    scratch_types=[
        pltpu.VMEM((B,), jnp.int32),
        pltpu.VMEM((CAPH + L,), jnp.int32),
        pltpu.VMEM((CAPH + L,), jnp.int32),
        pltpu.VMEM((CAPH + L,), jnp.int32),
        pltpu.VMEM((CAPH + L,), jnp.int32),
        pltpu.VMEM((2 * D, CW), jnp.float32),
        pltpu.VMEM((D, 2 * D), jnp.float32),
        pltpu.VMEM((RING * L, 2 * D), jnp.float32),
        pltpu.SemaphoreType.DMA,
        pltpu.SemaphoreType.DMA,
        pltpu.SemaphoreType.DMA,
    ],
)
def _extract_sc(w_i_hbm, w_j_hbm, wt_hbm, wtail_hbm, e_hbm,
                idx_v, hit_r, hit_p, ch_r, ch_p, chunk_v, wtail_v,
                stage_v, sem_c0, sem_c1, sem_s):
    wid = lax.axis_index("s") * NC + lax.axis_index("c")
    lane = lax.iota(jnp.int32, L)

    lo = wid * CPW * CW
    nch = jnp.minimum(CPW, NCHUNKS - wid * CPW)
    hi = jnp.where(wid == NW - 1, 1000000, lo + nch * CW)

    def scan_list(pos_base, off0):
        def body(i, off):
            vals = idx_v[pl.ds(i * L, L)]
            mask = (vals >= lo) & (vals < hi)

            pc = jnp.cumsum(mask.astype(jnp.int32))
            dst = off + pc - 1

            @pl.when(off <= CAPH - L)
            def _():
                plsc.store_scatter(hit_r, [dst], vals, mask=mask)
                plsc.store_scatter(hit_p, [dst], i * L + lane + pos_base,
                                   mask=mask)

            return jnp.where(off <= CAPH - L, off + pc[L - 1], off)

        return lax.fori_loop(0, B // L, body, off0)

    pltpu.sync_copy(w_i_hbm, idx_v)
    nh = scan_list(0, 0)
    pltpu.sync_copy(w_j_hbm, idx_v)
    nh = scan_list(B, nh)
    nhv = -(-nh // L)

    pltpu.sync_copy(wtail_hbm, wtail_v)

    def issue_chunk(c, parity):
        start = pl.multiple_of((wid * CPW + c) * CW, 128)
        sem = sem_c1 if parity else sem_c0
        pltpu.async_copy(wt_hbm.at[:, pl.ds(start, CW)],
                         chunk_v.at[pl.ds(parity * D, D)], sem)

    def drain_chunk(parity):
        sem = sem_c1 if parity else sem_c0
        pltpu.make_async_copy(
            wt_hbm.at[:, pl.ds(0, CW)], chunk_v.at[pl.ds(0, D)], sem).wait()

    def compact_window(clo, width):
        def body(v, off):
            vals = hit_r[pl.ds(v * L, L)]
            pos = hit_p[pl.ds(v * L, L)]
            mask = ((v * L + lane < nh)
                    & (vals >= clo) & (vals < clo + width))
            pc = jnp.cumsum(mask.astype(jnp.int32))
            dst = off + pc - 1
            plsc.store_scatter(ch_r, [dst], vals - clo, mask=mask)
            plsc.store_scatter(ch_p, [dst], pos, mask=mask)
            return off + pc[L - 1]

        return lax.fori_loop(0, nhv, body, 0)

    def extract_and_scatter(coff, gather_from):
        def body(h, posvec):
            slot = lax.rem(h, L)
            grp = lax.rem(lax.div(h, L), RING)
            cl = ch_r[pl.ds(h, L)][0]
            pos = ch_p[pl.ds(h, L)][0]
            row = grp * L + slot
            for k in range(4):
                stage_v[row, pl.ds(k * L, L)] = gather_from(k * L + lane, cl)
            posvec = jnp.where(lane == slot, pos, posvec)

            @pl.when(slot == L - 1)
            def _():
                pltpu.async_copy(stage_v.at[pl.ds(grp * L, L)],
                                 e_hbm.at[posvec], sem_s)
                pltpu.make_async_copy(stage_v.at[pl.ds(0, L)],
                                      e_hbm.at[pl.ds(0, L), :], sem_s).wait()

            return jnp.where(slot == L - 1, DUMP + lane, posvec)

        posvec = lax.fori_loop(0, coff, body, DUMP + lane)

        nfull = lax.div(coff, L)
        rem = lax.rem(coff, L)

        @pl.when(rem > 0)
        def _():
            grp = lax.rem(nfull, RING)
            pltpu.async_copy(stage_v.at[pl.ds(grp * L, L)],
                             e_hbm.at[posvec], sem_s)
            pltpu.make_async_copy(stage_v.at[pl.ds(0, L)],
                                  e_hbm.at[pl.ds(0, L), :], sem_s).wait()

    def pair_body(it, carry):
        for k in (0, 1):
            c = it * 2 + k

            @pl.when(c < nch)
            def _(c=c, k=k):
                issue_chunk(c, k)
                drain_chunk(k)
                coff = compact_window(lo + c * CW, CW)

                def gather_chunk(rowv, cl):
                    return plsc.load_gather(
                        chunk_v, [k * D + rowv, jnp.full((L,), cl, jnp.int32)])

                extract_and_scatter(coff, gather_chunk)

        return carry

    lax.fori_loop(0, (CPW + 1) // 2, pair_body, 0)

    @pl.when(wid == NW - 1)
    def _():
        coff = compact_window(jnp.int32(MAIN), jnp.int32(1000000 - MAIN))

        def gather_tail(rowv, cl):
            return plsc.load_gather(
                wtail_v, [rowv, jnp.full((L,), cl, jnp.int32)])

        extract_and_scatter(coff, gather_tail)


@functools.partial(
    pl.kernel,
    mesh=_mesh,
    compiler_params=_params,
    out_type=jax.ShapeDtypeStruct((B,), jnp.float32),
    scratch_types=[
        pltpu.VMEM((128, 2 * D), jnp.float32),
        pltpu.VMEM((128, 2 * D), jnp.float32),
        pltpu.VMEM((BPW,), jnp.float32),
        pltpu.SemaphoreType.DMA,
    ],
)
def _dot_sc(e_hbm, out_hbm, rows_i_v, rows_j_v, out_v, sem):
    wid = lax.axis_index("s") * NC + lax.axis_index("c")
    base = wid * BPW
    lane = lax.iota(jnp.int32, L)

    def block_body(t, carry):
        cp1 = pltpu.async_copy(
            e_hbm.at[pl.ds(base + t * 128, 128), :], rows_i_v, sem)
        cp2 = pltpu.async_copy(
            e_hbm.at[pl.ds(B + base + t * 128, 128), :], rows_j_v, sem)
        cp1.wait()
        cp2.wait()

        def group_body(g, carry2):
            acc = jnp.zeros((L,), jnp.float32)
            for b in range(L):
                rb = g * L + b
                s = (rows_i_v[rb, pl.ds(0, L)] * rows_j_v[rb, pl.ds(0, L)]
                     + rows_i_v[rb, pl.ds(L, L)] * rows_j_v[rb, pl.ds(L, L)]
                     + rows_i_v[rb, pl.ds(2 * L, L)] * rows_j_v[rb, pl.ds(2 * L, L)]
                     + rows_i_v[rb, pl.ds(3 * L, L)] * rows_j_v[rb, pl.ds(3 * L, L)])
                acc = jnp.where(lane == b, jnp.sum(s), acc)
            out_v[pl.ds(t * 128 + g * L, L)] = acc
            return carry2

        lax.fori_loop(0, 128 // L, group_body, 0)
        return carry

    lax.fori_loop(0, BPW // 128, block_body, 0)
    pltpu.sync_copy(out_v, out_hbm.at[pl.ds(base, BPW)])


def kernel(w_i, w_j, W):
    wt = W.T
    wtail = jnp.concatenate(
        [W[MAIN:].T, jnp.zeros((D, 2 * D - (1000000 - MAIN)), jnp.float32)],
        axis=1)
    e = _extract_sc(w_i.astype(jnp.int32), w_j.astype(jnp.int32), wt, wtail)
    out = _dot_sc(e)
    return out.reshape(B, 1)


# speedup vs baseline: 1.9475x; 1.0256x over previous
import functools

import jax
import jax.numpy as jnp
from jax import lax
from jax.experimental import pallas as pl
from jax.experimental.pallas import tpu as pltpu
from jax.experimental.pallas import tpu_sc as plsc

D = 64
B = 16384
NC = 2
NS = 16
L = 16
NW = NC * NS
BPW = B // NW
CW = 512
MAIN = 999936
NCHUNKS = MAIN // CW
CPW = -(-NCHUNKS // NW)
CAPH = 4096
RING = 8
EROWS = 2 * B + L
DUMP = 2 * B

_mesh = plsc.VectorSubcoreMesh(core_axis_name="c", subcore_axis_name="s")
_params = pltpu.CompilerParams(needs_layout_passes=False)


@functools.partial(
    pl.kernel,
    mesh=_mesh,
    compiler_params=_params,
    out_type=jax.ShapeDtypeStruct((EROWS, 2 * D), jnp.float32),
    scratch_types=[
        pltpu.VMEM((B,), jnp.int32),
        pltpu.VMEM((CAPH + L,), jnp.int32),
        pltpu.VMEM((CAPH + L,), jnp.int32),
        pltpu.VMEM((CAPH + L,), jnp.int32),
        pltpu.VMEM((CAPH + L,), jnp.int32),
        pltpu.VMEM((2 * D, CW), jnp.float32),
        pltpu.VMEM((D, 2 * D), jnp.float32),
        pltpu.VMEM((RING * L, 2 * D), jnp.float32),
        pltpu.SemaphoreType.DMA,
        pltpu.SemaphoreType.DMA,
        pltpu.SemaphoreType.DMA,
    ],
)
def _extract_sc(w_i_hbm, w_j_hbm, wt_hbm, wtail_hbm, e_hbm,
                idx_v, hit_r, hit_p, ch_r, ch_p, chunk_v, wtail_v,
                stage_v, sem_c0, sem_c1, sem_s):
    wid = lax.axis_index("s") * NC + lax.axis_index("c")
    lane = lax.iota(jnp.int32, L)

    lo = wid * CPW * CW
    nch = jnp.minimum(CPW, NCHUNKS - wid * CPW)
    hi = jnp.where(wid == NW - 1, 1000000, lo + nch * CW)

    def scan_list(pos_base, off0):
        def body(i, off):
            vals = idx_v[pl.ds(i * L, L)]
            mask = (vals >= lo) & (vals < hi)

            pc = jnp.cumsum(mask.astype(jnp.int32))
            dst = off + pc - 1

            @pl.when(off <= CAPH - L)
            def _():
                plsc.store_scatter(hit_r, [dst], vals, mask=mask)
                plsc.store_scatter(hit_p, [dst], i * L + lane + pos_base,
                                   mask=mask)

            return jnp.where(off <= CAPH - L, off + pc[L - 1], off)

        return lax.fori_loop(0, B // L, body, off0)

    pltpu.sync_copy(w_i_hbm, idx_v)
    nh = scan_list(0, 0)
    pltpu.sync_copy(w_j_hbm, idx_v)
    nh = scan_list(B, nh)
    nhv = -(-nh // L)

    pltpu.sync_copy(wtail_hbm, wtail_v)

    def issue_chunk(c, parity):
        start = pl.multiple_of((wid * CPW + c) * CW, 128)
        sem = sem_c1 if parity else sem_c0
        pltpu.async_copy(wt_hbm.at[:, pl.ds(start, CW)],
                         chunk_v.at[pl.ds(parity * D, D)], sem)

    def drain_chunk(parity):
        sem = sem_c1 if parity else sem_c0
        pltpu.make_async_copy(
            wt_hbm.at[:, pl.ds(0, CW)], chunk_v.at[pl.ds(0, D)], sem).wait()

    def compact_window(clo, width):
        def body(v, off):
            vals = hit_r[pl.ds(v * L, L)]
            pos = hit_p[pl.ds(v * L, L)]
            mask = ((v * L + lane < nh)
                    & (vals >= clo) & (vals < clo + width))
            pc = jnp.cumsum(mask.astype(jnp.int32))
            dst = off + pc - 1
            plsc.store_scatter(ch_r, [dst], vals - clo, mask=mask)
            plsc.store_scatter(ch_p, [dst], pos, mask=mask)
            return off + pc[L - 1]

        return lax.fori_loop(0, nhv, body, 0)

    def extract_and_scatter(coff, gather_from):
        def body(h, posvec):
            slot = lax.rem(h, L)
            grp = lax.rem(lax.div(h, L), RING)
            cl = ch_r[pl.ds(h, L)][0]
            pos = ch_p[pl.ds(h, L)][0]
            row = grp * L + slot
            for k in range(4):
                stage_v[row, pl.ds(k * L, L)] = gather_from(k * L + lane, cl)
            posvec = jnp.where(lane == slot, pos, posvec)

            @pl.when(slot == L - 1)
            def _():
                pltpu.async_copy(stage_v.at[pl.ds(grp * L, L)],
                                 e_hbm.at[posvec], sem_s)

            return jnp.where(slot == L - 1, DUMP + lane, posvec)

        posvec = lax.fori_loop(0, coff, body, DUMP + lane)

        nfull = lax.div(coff, L)
        rem = lax.rem(coff, L)

        @pl.when(rem > 0)
        def _():
            grp = lax.rem(nfull, RING)
            pltpu.async_copy(stage_v.at[pl.ds(grp * L, L)],
                             e_hbm.at[posvec], sem_s)

        nflush = jnp.minimum(nfull + jnp.where(rem > 0, 1, 0), RING)

        def drain(i, carry):
            pltpu.make_async_copy(stage_v.at[pl.ds(0, L)],
                                  e_hbm.at[pl.ds(0, L), :], sem_s).wait()
            return carry

        lax.fori_loop(0, nflush, drain, 0)

    issue_chunk(0, 0)

    def pair_body(it, carry):
        for k in (0, 1):
            c = it * 2 + k

            @pl.when(c < nch)
            def _(c=c, k=k):
                @pl.when(c + 1 < nch)
                def _():
                    issue_chunk(c + 1, 1 - k)

                drain_chunk(k)
                coff = compact_window(lo + c * CW, CW)

                def gather_chunk(rowv, cl):
                    return plsc.load_gather(
                        chunk_v, [k * D + rowv, jnp.full((L,), cl, jnp.int32)])

                extract_and_scatter(coff, gather_chunk)

        return carry

    lax.fori_loop(0, (CPW + 1) // 2, pair_body, 0)

    @pl.when(wid == NW - 1)
    def _():
        coff = compact_window(jnp.int32(MAIN), jnp.int32(1000000 - MAIN))

        def gather_tail(rowv, cl):
            return plsc.load_gather(
                wtail_v, [rowv, jnp.full((L,), cl, jnp.int32)])

        extract_and_scatter(coff, gather_tail)


@functools.partial(
    pl.kernel,
    mesh=_mesh,
    compiler_params=_params,
    out_type=jax.ShapeDtypeStruct((B,), jnp.float32),
    scratch_types=[
        pltpu.VMEM((128, 2 * D), jnp.float32),
        pltpu.VMEM((128, 2 * D), jnp.float32),
        pltpu.VMEM((BPW,), jnp.float32),
        pltpu.SemaphoreType.DMA,
    ],
)
def _dot_sc(e_hbm, out_hbm, rows_i_v, rows_j_v, out_v, sem):
    wid = lax.axis_index("s") * NC + lax.axis_index("c")
    base = wid * BPW
    lane = lax.iota(jnp.int32, L)

    def block_body(t, carry):
        cp1 = pltpu.async_copy(
            e_hbm.at[pl.ds(base + t * 128, 128), :], rows_i_v, sem)
        cp2 = pltpu.async_copy(
            e_hbm.at[pl.ds(B + base + t * 128, 128), :], rows_j_v, sem)
        cp1.wait()
        cp2.wait()

        def group_body(g, carry2):
            acc = jnp.zeros((L,), jnp.float32)
            for b in range(L):
                rb = g * L + b
                s = (rows_i_v[rb, pl.ds(0, L)] * rows_j_v[rb, pl.ds(0, L)]
                     + rows_i_v[rb, pl.ds(L, L)] * rows_j_v[rb, pl.ds(L, L)]
                     + rows_i_v[rb, pl.ds(2 * L, L)] * rows_j_v[rb, pl.ds(2 * L, L)]
                     + rows_i_v[rb, pl.ds(3 * L, L)] * rows_j_v[rb, pl.ds(3 * L, L)])
                acc = jnp.where(lane == b, jnp.sum(s), acc)
            out_v[pl.ds(t * 128 + g * L, L)] = acc
            return carry2

        lax.fori_loop(0, 128 // L, group_body, 0)
        return carry

    lax.fori_loop(0, BPW // 128, block_body, 0)
    pltpu.sync_copy(out_v, out_hbm.at[pl.ds(base, BPW)])


def kernel(w_i, w_j, W):
    wt = W.T
    wtail = jnp.concatenate(
        [W[MAIN:].T, jnp.zeros((D, 2 * D - (1000000 - MAIN)), jnp.float32)],
        axis=1)
    e = _extract_sc(w_i.astype(jnp.int32), w_j.astype(jnp.int32), wt, wtail)
    out = _dot_sc(e)
    return out.reshape(B, 1)
